# Initial kernel scaffold; baseline (speedup 1.0000x reference)
import jax
import jax.numpy as jnp
from jax.experimental import pallas as pl


def kernel(input, target):
    raise NotImplementedError("write your pallas kernel here")


# speedup vs baseline: 25.2961x; 25.2961x over previous
import functools

import jax
import jax.numpy as jnp
from jax import lax
from jax.experimental import pallas as pl
from jax.experimental.pallas import tpu as pltpu
from jax.experimental.pallas import tpu_sc as plsc

N_TOTAL = 16 * 1 * 512 * 512
K_BINS = 8192
NUM_WORKERS = 32
ELEMS_PER_TILE = N_TOTAL // NUM_WORKERS
CHUNK = 8192
NCHUNKS = ELEMS_PER_TILE // CHUNK
LANES = 16
UNROLL = 4


def _phase1_body(x_hbm, t_hbm, cnt_hbm, sum_hbm, s1_hbm, pa_hbm,
                 xbuf, tbuf, cnt_v, sum_v, s1_v, pa_v):
    wid = lax.axis_index("s") * 2 + lax.axis_index("c")
    base = wid * ELEMS_PER_TILE

    zeros = jnp.zeros((LANES,), jnp.float32)

    def zero_body(i, c):
        cnt_v[pl.ds(i * LANES, LANES)] = zeros
        sum_v[pl.ds(i * LANES, LANES)] = zeros
        return c

    lax.fori_loop(0, K_BINS // LANES, zero_body, 0)
    s1_v[...] = zeros
    pa_v[...] = zeros

    ones = jnp.ones((LANES,), jnp.float32)

    def chunk_body(ci, c):
        off = base + ci * CHUNK
        pltpu.sync_copy(x_hbm.at[pl.ds(off, CHUNK)], xbuf)
        pltpu.sync_copy(t_hbm.at[pl.ds(off, CHUNK)], tbuf)

        def vec_body(i, c2):
            for u in range(UNROLL):
                o = (i * UNROLL + u) * LANES
                xv = xbuf[pl.ds(o, LANES)]
                tv = tbuf[pl.ds(o, LANES)]
                p = 1.0 / (1.0 + jnp.exp(-xv))
                tf = tv.astype(jnp.float32)
                s1_v[...] = s1_v[...] + p * tf
                pa_v[...] = pa_v[...] + tf
                m = tv == 0
                b = jnp.minimum((p * K_BINS).astype(jnp.int32), K_BINS - 1)
                plsc.addupdate_scatter(cnt_v, [b], ones, mask=m)
                plsc.addupdate_scatter(sum_v, [b], p, mask=m)
            return c2

        lax.fori_loop(0, CHUNK // (LANES * UNROLL), vec_body, c)
        return c

    lax.fori_loop(0, NCHUNKS, chunk_body, 0)

    pltpu.sync_copy(cnt_v, cnt_hbm.at[wid])
    pltpu.sync_copy(sum_v, sum_hbm.at[wid])
    pltpu.sync_copy(s1_v, s1_hbm.at[wid])
    pltpu.sync_copy(pa_v, pa_hbm.at[wid])


def _phase1(x_flat, t_flat):
    mesh = plsc.VectorSubcoreMesh(core_axis_name="c", subcore_axis_name="s")
    f = functools.partial(
        pl.kernel,
        mesh=mesh,
        compiler_params=pltpu.CompilerParams(needs_layout_passes=False),
        out_type=[
            jax.ShapeDtypeStruct((NUM_WORKERS, K_BINS), jnp.float32),
            jax.ShapeDtypeStruct((NUM_WORKERS, K_BINS), jnp.float32),
            jax.ShapeDtypeStruct((NUM_WORKERS, LANES), jnp.float32),
            jax.ShapeDtypeStruct((NUM_WORKERS, LANES), jnp.float32),
        ],
        scratch_types=[
            pltpu.VMEM((CHUNK,), jnp.float32),
            pltpu.VMEM((CHUNK,), jnp.int32),
            pltpu.VMEM((K_BINS,), jnp.float32),
            pltpu.VMEM((K_BINS,), jnp.float32),
            pltpu.VMEM((LANES,), jnp.float32),
            pltpu.VMEM((LANES,), jnp.float32),
        ],
    )(_phase1_body)
    return f(x_flat, t_flat)


def _phase2_body(cnt_ref, sum_ref, s1_ref, pa_ref, out_ref):
    cnt = jnp.sum(cnt_ref[...], axis=0)
    sm = jnp.sum(sum_ref[...], axis=0)
    s1 = jnp.sum(s1_ref[...])
    pos = jnp.sum(pa_ref[...])

    r128 = lax.broadcasted_iota(jnp.int32, (128, 128), 0)
    c128 = lax.broadcasted_iota(jnp.int32, (128, 128), 1)
    tri_incl = (r128 <= c128).astype(jnp.float32)
    r64 = lax.broadcasted_iota(jnp.int32, (64, 64), 0)
    c64 = lax.broadcasted_iota(jnp.int32, (64, 64), 1)
    tri_strict = (c64 < r64).astype(jnp.float32)

    within = jnp.dot(cnt, tri_incl, preferred_element_type=jnp.float32)
    prev_rows = jnp.dot(tri_strict, cnt, preferred_element_type=jnp.float32)
    row_off = jnp.sum(prev_rows, axis=1, keepdims=True)
    csum = within + row_off

    total_neg = jnp.sum(cnt)
    rank = total_neg - csum
    contrib = sm * pos / ((pos + rank) * (pos + rank + cnt))
    loss = 1.0 - s1 / N_TOTAL + jnp.sum(contrib)
    out_ref[0, 0] = jnp.where(pos > 0, loss, 0.0)


def _phase2(cnt_part, sum_part, s1_part, pa_part, interpret=False):
    return pl.pallas_call(
        _phase2_body,
        out_shape=jax.ShapeDtypeStruct((1, 1), jnp.float32),
        out_specs=pl.BlockSpec(memory_space=pltpu.SMEM),
        interpret=interpret,
    )(cnt_part, sum_part, s1_part, pa_part)


def kernel(input, target):
    x = input.reshape(N_TOTAL)
    t = target.reshape(N_TOTAL).astype(jnp.int32)
    cnt_part, sum_part, s1_part, pa_part = _phase1(x, t)
    out = _phase2(
        cnt_part.reshape(NUM_WORKERS, 64, 128),
        sum_part.reshape(NUM_WORKERS, 64, 128),
        s1_part,
        pa_part,
    )
    return out[0, 0]


# speedup vs baseline: 26.1667x; 1.0344x over previous
import functools

import jax
import jax.numpy as jnp
from jax import lax
from jax.experimental import pallas as pl
from jax.experimental.pallas import tpu as pltpu
from jax.experimental.pallas import tpu_sc as plsc

N_TOTAL = 16 * 1 * 512 * 512
K_BINS = 8192
NBINS2 = 2 * K_BINS
NUM_WORKERS = 32
ELEMS_PER_TILE = N_TOTAL // NUM_WORKERS
CHUNK = 8192
NCHUNKS = ELEMS_PER_TILE // CHUNK
LANES = 16
UNROLL = 8


def _phase1_body(x_hbm, t_hbm, cnt_hbm, sum_hbm,
                 xbuf0, tbuf0, xbuf1, tbuf1, cnt_v, sum_v,
                 sem_x0, sem_t0, sem_x1, sem_t1):
    wid = lax.axis_index("s") * 2 + lax.axis_index("c")
    base = wid * ELEMS_PER_TILE

    zeros = jnp.zeros((LANES,), jnp.float32)

    def zero_body(i, c):
        cnt_v[pl.ds(i * LANES, LANES)] = zeros
        sum_v[pl.ds(i * LANES, LANES)] = zeros
        return c

    lax.fori_loop(0, NBINS2 // LANES, zero_body, 0)

    ones = jnp.ones((LANES,), jnp.float32)
    xbufs = (xbuf0, xbuf1)
    tbufs = (tbuf0, tbuf1)
    sems = ((sem_x0, sem_t0), (sem_x1, sem_t1))

    def start(ci, slot):
        off = base + ci * CHUNK
        cx = pltpu.make_async_copy(x_hbm.at[pl.ds(off, CHUNK)], xbufs[slot],
                                   sems[slot][0])
        ct = pltpu.make_async_copy(t_hbm.at[pl.ds(off, CHUNK)], tbufs[slot],
                                   sems[slot][1])
        cx.start()
        ct.start()
        return cx, ct

    def process(slot):
        xbuf, tbuf = xbufs[slot], tbufs[slot]

        def vec_body(i, c2):
            for u in range(UNROLL):
                o = (i * UNROLL + u) * LANES
                xv = xbuf[pl.ds(o, LANES)]
                tv = tbuf[pl.ds(o, LANES)]
                p = 1.0 / (1.0 + jnp.exp(-xv))
                b = jnp.minimum((p * K_BINS).astype(jnp.int32), K_BINS - 1)
                b = b + tv * K_BINS
                plsc.addupdate_scatter(cnt_v, [b], ones)
                plsc.addupdate_scatter(sum_v, [b], p)
            return c2

        lax.fori_loop(0, CHUNK // (LANES * UNROLL), vec_body, 0)

    pending = start(0, 0)
    for ci in range(NCHUNKS):
        slot = ci % 2
        pending[0].wait()
        pending[1].wait()
        if ci + 1 < NCHUNKS:
            pending = start(ci + 1, 1 - slot)
        process(slot)

    pltpu.sync_copy(cnt_v, cnt_hbm.at[wid])
    pltpu.sync_copy(sum_v, sum_hbm.at[wid])


def _phase1(x_flat, t_flat):
    mesh = plsc.VectorSubcoreMesh(core_axis_name="c", subcore_axis_name="s")
    f = functools.partial(
        pl.kernel,
        mesh=mesh,
        compiler_params=pltpu.CompilerParams(needs_layout_passes=False),
        out_type=[
            jax.ShapeDtypeStruct((NUM_WORKERS, NBINS2), jnp.float32),
            jax.ShapeDtypeStruct((NUM_WORKERS, NBINS2), jnp.float32),
        ],
        scratch_types=[
            pltpu.VMEM((CHUNK,), jnp.float32),
            pltpu.VMEM((CHUNK,), jnp.int32),
            pltpu.VMEM((CHUNK,), jnp.float32),
            pltpu.VMEM((CHUNK,), jnp.int32),
            pltpu.VMEM((NBINS2,), jnp.float32),
            pltpu.VMEM((NBINS2,), jnp.float32),
            pltpu.SemaphoreType.DMA,
            pltpu.SemaphoreType.DMA,
            pltpu.SemaphoreType.DMA,
            pltpu.SemaphoreType.DMA,
        ],
    )(_phase1_body)
    return f(x_flat, t_flat)


def _phase2_body(cnt_ref, sum_ref, out_ref):
    cnt = jnp.sum(cnt_ref[...], axis=0)
    sm = jnp.sum(sum_ref[...], axis=0)

    rr = lax.broadcasted_iota(jnp.int32, (128, 128), 0)
    cc = lax.broadcasted_iota(jnp.int32, (128, 128), 1)
    neg_row = (rr < 64).astype(jnp.float32)
    tri_incl = (rr <= cc).astype(jnp.float32)
    tri_strict = (cc < rr).astype(jnp.float32)

    cnt_neg = cnt * neg_row
    pos = jnp.sum(cnt * (1.0 - neg_row))
    s1 = jnp.sum(sm * (1.0 - neg_row))

    within = jnp.dot(cnt_neg, tri_incl, preferred_element_type=jnp.float32)
    prev_rows = jnp.dot(tri_strict, cnt_neg, preferred_element_type=jnp.float32)
    row_off = jnp.sum(prev_rows, axis=1, keepdims=True)
    csum = within + row_off

    total_neg = jnp.sum(cnt_neg)
    rank = total_neg - csum
    contrib = sm * neg_row * pos / ((pos + rank) * (pos + rank + cnt_neg))
    loss = 1.0 - s1 / N_TOTAL + jnp.sum(contrib)
    out_ref[0, 0] = jnp.where(pos > 0, loss, 0.0)


def _phase2(cnt_part, sum_part, interpret=False):
    return pl.pallas_call(
        _phase2_body,
        out_shape=jax.ShapeDtypeStruct((1, 1), jnp.float32),
        out_specs=pl.BlockSpec(memory_space=pltpu.SMEM),
        interpret=interpret,
    )(cnt_part, sum_part)


def kernel(input, target):
    x = input.reshape(N_TOTAL)
    t = target.reshape(N_TOTAL).astype(jnp.int32)
    cnt_part, sum_part = _phase1(x, t)
    out = _phase2(
        cnt_part.reshape(NUM_WORKERS, 128, 128),
        sum_part.reshape(NUM_WORKERS, 128, 128),
    )
    return out[0, 0]


# speedup vs baseline: 63.5835x; 2.4299x over previous
import functools

import jax
import jax.numpy as jnp
from jax import lax
from jax.experimental import pallas as pl
from jax.experimental.pallas import tpu as pltpu
from jax.experimental.pallas import tpu_sc as plsc

N_TOTAL = 16 * 1 * 512 * 512
K_BINS = 8192
NBINS2 = 2 * K_BINS
NUM_WORKERS = 32
ELEMS_PER_TILE = N_TOTAL // NUM_WORKERS
CHUNK = 8192
NCHUNKS = ELEMS_PER_TILE // CHUNK
LANES = 16
UNROLL = 8


def _phase1_body(x_hbm, t_hbm, cnt_hbm, sum_hbm,
                 xbuf0, tbuf0, xbuf1, tbuf1, cnt_v, sum_v,
                 sem_x0, sem_t0, sem_x1, sem_t1):
    wid = lax.axis_index("s") * 2 + lax.axis_index("c")
    base = wid * ELEMS_PER_TILE

    zeros = jnp.zeros((LANES,), jnp.float32)

    def zero_body(i, c):
        cnt_v[pl.ds(i * LANES, LANES)] = zeros
        sum_v[pl.ds(i * LANES, LANES)] = zeros
        return c

    lax.fori_loop(0, NBINS2 // LANES, zero_body, 0)

    ones = jnp.ones((LANES,), jnp.float32)
    xbufs = (xbuf0, xbuf1)
    tbufs = (tbuf0, tbuf1)
    sems = ((sem_x0, sem_t0), (sem_x1, sem_t1))

    def start(ci, slot):
        off = base + ci * CHUNK
        cx = pltpu.make_async_copy(x_hbm.at[pl.ds(off, CHUNK)], xbufs[slot],
                                   sems[slot][0])
        ct = pltpu.make_async_copy(t_hbm.at[pl.ds(off, CHUNK)], tbufs[slot],
                                   sems[slot][1])
        cx.start()
        ct.start()
        return cx, ct

    def process(slot):
        xbuf, tbuf = xbufs[slot], tbufs[slot]

        def vec_body(i, c2):
            offs = [0] * UNROLL
            xs = [None] * UNROLL
            ts = [None] * UNROLL
            for u in range(UNROLL):
                o = (i * UNROLL + u) * LANES
                xs[u] = xbuf[pl.ds(o, LANES)]
                ts[u] = tbuf[pl.ds(o, LANES)]
            es = [jnp.exp(-xs[u]) for u in range(UNROLL)]
            ds = [1.0 + es[u] for u in range(UNROLL)]
            ps = [1.0 / ds[u] for u in range(UNROLL)]
            bs = [
                (ps[u] * (K_BINS - 0.5)).astype(jnp.int32) + ts[u] * K_BINS
                for u in range(UNROLL)
            ]
            for u in range(UNROLL):
                plsc.addupdate_scatter(cnt_v, [bs[u]], ones)
                plsc.addupdate_scatter(sum_v, [bs[u]], ps[u])
            return c2

        lax.fori_loop(0, CHUNK // (LANES * UNROLL), vec_body, 0)

    pending = start(0, 0)
    for ci in range(NCHUNKS):
        slot = ci % 2
        pending[0].wait()
        pending[1].wait()
        if ci + 1 < NCHUNKS:
            pending = start(ci + 1, 1 - slot)
        process(slot)

    pltpu.sync_copy(cnt_v, cnt_hbm.at[wid])
    pltpu.sync_copy(sum_v, sum_hbm.at[wid])


def _phase1(x_flat, t_flat):
    mesh = plsc.VectorSubcoreMesh(core_axis_name="c", subcore_axis_name="s")
    f = functools.partial(
        pl.kernel,
        mesh=mesh,
        compiler_params=pltpu.CompilerParams(needs_layout_passes=False),
        out_type=[
            jax.ShapeDtypeStruct((NUM_WORKERS, NBINS2), jnp.float32),
            jax.ShapeDtypeStruct((NUM_WORKERS, NBINS2), jnp.float32),
        ],
        scratch_types=[
            pltpu.VMEM((CHUNK,), jnp.float32),
            pltpu.VMEM((CHUNK,), jnp.int32),
            pltpu.VMEM((CHUNK,), jnp.float32),
            pltpu.VMEM((CHUNK,), jnp.int32),
            pltpu.VMEM((NBINS2,), jnp.float32),
            pltpu.VMEM((NBINS2,), jnp.float32),
            pltpu.SemaphoreType.DMA,
            pltpu.SemaphoreType.DMA,
            pltpu.SemaphoreType.DMA,
            pltpu.SemaphoreType.DMA,
        ],
    )(_phase1_body)
    return f(x_flat, t_flat)


def _phase2_body(cnt_ref, sum_ref, out_ref):
    cnt = jnp.sum(cnt_ref[...], axis=0)
    sm = jnp.sum(sum_ref[...], axis=0)

    rr = lax.broadcasted_iota(jnp.int32, (128, 128), 0)
    cc = lax.broadcasted_iota(jnp.int32, (128, 128), 1)
    neg_row = (rr < 64).astype(jnp.float32)
    tri_incl = (rr <= cc).astype(jnp.float32)
    tri_strict = (cc < rr).astype(jnp.float32)

    cnt_neg = cnt * neg_row
    pos = jnp.sum(cnt * (1.0 - neg_row))
    s1 = jnp.sum(sm * (1.0 - neg_row))

    within = jnp.dot(cnt_neg, tri_incl, preferred_element_type=jnp.float32)
    prev_rows = jnp.dot(tri_strict, cnt_neg, preferred_element_type=jnp.float32)
    row_off = jnp.sum(prev_rows, axis=1, keepdims=True)
    csum = within + row_off

    total_neg = jnp.sum(cnt_neg)
    rank = total_neg - csum
    contrib = sm * neg_row * pos / ((pos + rank) * (pos + rank + cnt_neg))
    loss = 1.0 - s1 / N_TOTAL + jnp.sum(contrib)
    out_ref[0, 0] = jnp.where(pos > 0, loss, 0.0)


def _phase2(cnt_part, sum_part, interpret=False):
    return pl.pallas_call(
        _phase2_body,
        out_shape=jax.ShapeDtypeStruct((1, 1), jnp.float32),
        out_specs=pl.BlockSpec(memory_space=pltpu.SMEM),
        interpret=interpret,
    )(cnt_part, sum_part)


def kernel(input, target):
    x = input.reshape(N_TOTAL)
    t = target.reshape(N_TOTAL).astype(jnp.int32)
    cnt_part, sum_part = _phase1(x, t)
    out = _phase2(
        cnt_part.reshape(NUM_WORKERS, 128, 128),
        sum_part.reshape(NUM_WORKERS, 128, 128),
    )
    return out[0, 0]


# speedup vs baseline: 87.8934x; 1.3823x over previous
import functools

import jax
import jax.numpy as jnp
from jax import lax
from jax.experimental import pallas as pl
from jax.experimental.pallas import tpu as pltpu
from jax.experimental.pallas import tpu_sc as plsc

N_TOTAL = 16 * 1 * 512 * 512
K_BINS = 8192
NBINS2 = 2 * K_BINS
NUM_WORKERS = 32
ROWS = 8192
COLS = 512
ROWS_PER_TILE = ROWS // NUM_WORKERS
CHUNK_ROWS = 16
CHUNK = CHUNK_ROWS * COLS
NCHUNKS = ROWS_PER_TILE // CHUNK_ROWS
LANES = 16
UNROLL = 8


def _phase1_body(x_hbm, t_hbm, cnt_hbm, sum_hbm,
                 xbuf0, tbuf0, xbuf1, tbuf1, cnt_v, sum_v,
                 sem_x0, sem_t0, sem_x1, sem_t1):
    wid = lax.axis_index("s") * 2 + lax.axis_index("c")
    base = wid * ROWS_PER_TILE

    zeros = jnp.zeros((LANES,), jnp.float32)

    def zero_body(i, c):
        cnt_v[pl.ds(i * LANES, LANES)] = zeros
        sum_v[pl.ds(i * LANES, LANES)] = zeros
        return c

    lax.fori_loop(0, NBINS2 // LANES, zero_body, 0)

    ones = jnp.ones((LANES,), jnp.float32)
    xbufs = (xbuf0, xbuf1)
    tbufs = (tbuf0, tbuf1)
    sems = ((sem_x0, sem_t0), (sem_x1, sem_t1))

    def start(ci, slot):
        row0 = base + ci * CHUNK_ROWS
        cx = pltpu.make_async_copy(x_hbm.at[pl.ds(row0, CHUNK_ROWS), :],
                                   xbufs[slot], sems[slot][0])
        ct = pltpu.make_async_copy(t_hbm.at[pl.ds(row0, CHUNK_ROWS), :],
                                   tbufs[slot], sems[slot][1])
        cx.start()
        ct.start()
        return cx, ct

    def process(slot):
        xbuf, tbuf = xbufs[slot], tbufs[slot]
        vec_per_row = COLS // LANES
        groups_per_row = vec_per_row // UNROLL

        def vec_body(i, c2):
            r = i // groups_per_row
            cbase = (i % groups_per_row) * (UNROLL * LANES)
            xs = [None] * UNROLL
            ts = [None] * UNROLL
            for u in range(UNROLL):
                o = cbase + u * LANES
                xs[u] = xbuf[r, pl.ds(o, LANES)]
                ts[u] = tbuf[r, pl.ds(o, LANES)]
            es = [jnp.exp(-xs[u]) for u in range(UNROLL)]
            ds = [1.0 + es[u] for u in range(UNROLL)]
            ps = [1.0 / ds[u] for u in range(UNROLL)]
            bs = [
                (ps[u] * (K_BINS - 0.5)).astype(jnp.int32) + ts[u] * K_BINS
                for u in range(UNROLL)
            ]
            for u in range(UNROLL):
                plsc.addupdate_scatter(cnt_v, [bs[u]], ones)
                plsc.addupdate_scatter(sum_v, [bs[u]], ps[u])
            return c2

        lax.fori_loop(0, CHUNK_ROWS * groups_per_row, vec_body, 0)

    pending = start(0, 0)
    for ci in range(NCHUNKS):
        slot = ci % 2
        pending[0].wait()
        pending[1].wait()
        if ci + 1 < NCHUNKS:
            pending = start(ci + 1, 1 - slot)
        process(slot)

    pltpu.sync_copy(cnt_v, cnt_hbm.at[wid])
    pltpu.sync_copy(sum_v, sum_hbm.at[wid])


def _phase1(x_flat, t_flat):
    mesh = plsc.VectorSubcoreMesh(core_axis_name="c", subcore_axis_name="s")
    f = functools.partial(
        pl.kernel,
        mesh=mesh,
        compiler_params=pltpu.CompilerParams(needs_layout_passes=False,
                                             use_tc_tiling_on_sc=True),
        out_type=[
            jax.ShapeDtypeStruct((NUM_WORKERS, NBINS2), jnp.float32),
            jax.ShapeDtypeStruct((NUM_WORKERS, NBINS2), jnp.float32),
        ],
        scratch_types=[
            pltpu.VMEM((CHUNK_ROWS, COLS), jnp.float32),
            pltpu.VMEM((CHUNK_ROWS, COLS), jnp.int32),
            pltpu.VMEM((CHUNK_ROWS, COLS), jnp.float32),
            pltpu.VMEM((CHUNK_ROWS, COLS), jnp.int32),
            pltpu.VMEM((NBINS2,), jnp.float32),
            pltpu.VMEM((NBINS2,), jnp.float32),
            pltpu.SemaphoreType.DMA,
            pltpu.SemaphoreType.DMA,
            pltpu.SemaphoreType.DMA,
            pltpu.SemaphoreType.DMA,
        ],
    )(_phase1_body)
    return f(x_flat, t_flat)


def _phase2_body(cnt_ref, sum_ref, out_ref):
    cnt = jnp.sum(cnt_ref[...], axis=0)
    sm = jnp.sum(sum_ref[...], axis=0)

    rr = lax.broadcasted_iota(jnp.int32, (128, 128), 0)
    cc = lax.broadcasted_iota(jnp.int32, (128, 128), 1)
    neg_row = (rr < 64).astype(jnp.float32)
    tri_incl = (rr <= cc).astype(jnp.float32)
    tri_strict = (cc < rr).astype(jnp.float32)

    cnt_neg = cnt * neg_row
    pos = jnp.sum(cnt * (1.0 - neg_row))
    s1 = jnp.sum(sm * (1.0 - neg_row))

    within = jnp.dot(cnt_neg, tri_incl, preferred_element_type=jnp.float32)
    prev_rows = jnp.dot(tri_strict, cnt_neg, preferred_element_type=jnp.float32)
    row_off = jnp.sum(prev_rows, axis=1, keepdims=True)
    csum = within + row_off

    total_neg = jnp.sum(cnt_neg)
    rank = total_neg - csum
    contrib = sm * neg_row * pos / ((pos + rank) * (pos + rank + cnt_neg))
    loss = 1.0 - s1 / N_TOTAL + jnp.sum(contrib)
    out_ref[0, 0] = jnp.where(pos > 0, loss, 0.0)


def _phase2(cnt_part, sum_part, interpret=False):
    return pl.pallas_call(
        _phase2_body,
        out_shape=jax.ShapeDtypeStruct((1, 1), jnp.float32),
        out_specs=pl.BlockSpec(memory_space=pltpu.SMEM),
        interpret=interpret,
    )(cnt_part, sum_part)


def kernel(input, target):
    x = input.reshape(ROWS, COLS)
    t = target.reshape(ROWS, COLS).astype(jnp.int32)
    cnt_part, sum_part = _phase1(x, t)
    out = _phase2(
        cnt_part.reshape(NUM_WORKERS, 128, 128),
        sum_part.reshape(NUM_WORKERS, 128, 128),
    )
    return out[0, 0]


# speedup vs baseline: 102.7435x; 1.1690x over previous
import functools

import jax
import jax.numpy as jnp
from jax import lax
from jax.experimental import pallas as pl
from jax.experimental.pallas import tpu as pltpu
from jax.experimental.pallas import tpu_sc as plsc

N_TOTAL = 16 * 1 * 512 * 512
K_BINS = 8192
NBINS2 = 2 * K_BINS
NUM_WORKERS = 32
ROWS = 8192
COLS = 512
ROWS_PER_TILE = ROWS // NUM_WORKERS
CHUNK_ROWS = 16
CHUNK = CHUNK_ROWS * COLS
NCHUNKS = ROWS_PER_TILE // CHUNK_ROWS
LANES = 16
UNROLL = 8


def _phase1_body(x_hbm, t_hbm, cnt_hbm,
                 xbuf0, tbuf0, xbuf1, tbuf1, cnt_v,
                 sem_x0, sem_t0, sem_x1, sem_t1):
    wid = lax.axis_index("s") * 2 + lax.axis_index("c")
    base = wid * ROWS_PER_TILE

    zeros = jnp.zeros((LANES,), jnp.float32)

    def zero_body(i, c):
        cnt_v[pl.ds(i * LANES, LANES)] = zeros
        return c

    lax.fori_loop(0, NBINS2 // LANES, zero_body, 0)

    ones = jnp.ones((LANES,), jnp.float32)
    xbufs = (xbuf0, xbuf1)
    tbufs = (tbuf0, tbuf1)
    sems = ((sem_x0, sem_t0), (sem_x1, sem_t1))

    def start(ci, slot):
        row0 = base + ci * CHUNK_ROWS
        cx = pltpu.make_async_copy(x_hbm.at[pl.ds(row0, CHUNK_ROWS), :],
                                   xbufs[slot], sems[slot][0])
        ct = pltpu.make_async_copy(t_hbm.at[pl.ds(row0, CHUNK_ROWS), :],
                                   tbufs[slot], sems[slot][1])
        cx.start()
        ct.start()
        return cx, ct

    def process(slot):
        xbuf, tbuf = xbufs[slot], tbufs[slot]
        vec_per_row = COLS // LANES
        groups_per_row = vec_per_row // UNROLL

        def vec_body(i, c2):
            r = i // groups_per_row
            cbase = (i % groups_per_row) * (UNROLL * LANES)
            xs = [None] * UNROLL
            ts = [None] * UNROLL
            for u in range(UNROLL):
                o = cbase + u * LANES
                xs[u] = xbuf[r, pl.ds(o, LANES)]
                ts[u] = tbuf[r, pl.ds(o, LANES)]
            es = [jnp.exp(-xs[u]) for u in range(UNROLL)]
            ds = [1.0 + es[u] for u in range(UNROLL)]
            ps = [1.0 / ds[u] for u in range(UNROLL)]
            bs = [
                (ps[u] * (K_BINS - 0.5)).astype(jnp.int32) + ts[u] * K_BINS
                for u in range(UNROLL)
            ]
            for u in range(UNROLL):
                plsc.addupdate_scatter(cnt_v, [bs[u]], ones)
            return c2

        lax.fori_loop(0, CHUNK_ROWS * groups_per_row, vec_body, 0)

    pending = start(0, 0)
    for ci in range(NCHUNKS):
        slot = ci % 2
        pending[0].wait()
        pending[1].wait()
        if ci + 1 < NCHUNKS:
            pending = start(ci + 1, 1 - slot)
        process(slot)

    pltpu.sync_copy(cnt_v, cnt_hbm.at[wid])


def _phase1(x_flat, t_flat):
    mesh = plsc.VectorSubcoreMesh(core_axis_name="c", subcore_axis_name="s")
    f = functools.partial(
        pl.kernel,
        mesh=mesh,
        compiler_params=pltpu.CompilerParams(needs_layout_passes=False,
                                             use_tc_tiling_on_sc=True),
        out_type=[
            jax.ShapeDtypeStruct((NUM_WORKERS, NBINS2), jnp.float32),
        ],
        scratch_types=[
            pltpu.VMEM((CHUNK_ROWS, COLS), jnp.float32),
            pltpu.VMEM((CHUNK_ROWS, COLS), jnp.int32),
            pltpu.VMEM((CHUNK_ROWS, COLS), jnp.float32),
            pltpu.VMEM((CHUNK_ROWS, COLS), jnp.int32),
            pltpu.VMEM((NBINS2,), jnp.float32),
            pltpu.SemaphoreType.DMA,
            pltpu.SemaphoreType.DMA,
            pltpu.SemaphoreType.DMA,
            pltpu.SemaphoreType.DMA,
        ],
    )(_phase1_body)
    return f(x_flat, t_flat)


def _phase2_body(cnt_ref, out_ref):
    cnt = jnp.sum(cnt_ref[...], axis=0)

    rr = lax.broadcasted_iota(jnp.int32, (128, 128), 0)
    cc = lax.broadcasted_iota(jnp.int32, (128, 128), 1)
    g = rr * 128 + cc
    b = jnp.where(g >= K_BINS, g - K_BINS, g).astype(jnp.float32)
    mid = (b + 0.5) * (1.0 / (K_BINS - 0.5))
    sm = cnt * mid

    neg_row = (rr < 64).astype(jnp.float32)
    tri_incl = (rr <= cc).astype(jnp.float32)
    tri_strict = (cc < rr).astype(jnp.float32)

    cnt_neg = cnt * neg_row
    pos = jnp.sum(cnt * (1.0 - neg_row))
    s1 = jnp.sum(sm * (1.0 - neg_row))

    within = jnp.dot(cnt_neg, tri_incl, preferred_element_type=jnp.float32)
    prev_rows = jnp.dot(tri_strict, cnt_neg, preferred_element_type=jnp.float32)
    row_off = jnp.sum(prev_rows, axis=1, keepdims=True)
    csum = within + row_off

    total_neg = jnp.sum(cnt_neg)
    rank = total_neg - csum
    contrib = sm * neg_row * pos / ((pos + rank) * (pos + rank + cnt_neg))
    loss = 1.0 - s1 / N_TOTAL + jnp.sum(contrib)
    out_ref[0, 0] = jnp.where(pos > 0, loss, 0.0)


def _phase2(cnt_part, interpret=False):
    return pl.pallas_call(
        _phase2_body,
        out_shape=jax.ShapeDtypeStruct((1, 1), jnp.float32),
        out_specs=pl.BlockSpec(memory_space=pltpu.SMEM),
        interpret=interpret,
    )(cnt_part)


def kernel(input, target):
    x = input.reshape(ROWS, COLS)
    t = target.reshape(ROWS, COLS).astype(jnp.int32)
    (cnt_part,) = _phase1(x, t)
    out = _phase2(cnt_part.reshape(NUM_WORKERS, 128, 128))
    return out[0, 0]


# speedup vs baseline: 104.7726x; 1.0197x over previous
import functools

import jax
import jax.numpy as jnp
from jax import lax
from jax.experimental import pallas as pl
from jax.experimental.pallas import tpu as pltpu
from jax.experimental.pallas import tpu_sc as plsc

N_TOTAL = 16 * 1 * 512 * 512
K_BINS = 8192
NBINS2 = 2 * K_BINS
NUM_WORKERS = 32
ROWS = 8192
COLS = 512
ROWS_PER_TILE = ROWS // NUM_WORKERS
CHUNK_ROWS = 16
CHUNK = CHUNK_ROWS * COLS
NCHUNKS = ROWS_PER_TILE // CHUNK_ROWS
LANES = 16
UNROLL = 8


def _phase1_body(x_hbm, t_hbm, cnt_hbm,
                 xbuf0, tbuf0, xbuf1, tbuf1, cnt_v,
                 sem_x0, sem_t0, sem_x1, sem_t1):
    wid = lax.axis_index("s") * 2 + lax.axis_index("c")
    base = wid * ROWS_PER_TILE

    zeros = jnp.zeros((LANES,), jnp.float32)

    def zero_body(i, c):
        cnt_v[pl.ds(i * LANES, LANES)] = zeros
        return c

    lax.fori_loop(0, NBINS2 // LANES, zero_body, 0)

    ones = jnp.ones((LANES,), jnp.float32)
    xbufs = (xbuf0, xbuf1)
    tbufs = (tbuf0, tbuf1)
    sems = ((sem_x0, sem_t0), (sem_x1, sem_t1))

    def start(ci, slot):
        row0 = base + ci * CHUNK_ROWS
        cx = pltpu.make_async_copy(x_hbm.at[pl.ds(row0, CHUNK_ROWS), :],
                                   xbufs[slot], sems[slot][0])
        ct = pltpu.make_async_copy(t_hbm.at[pl.ds(row0, CHUNK_ROWS), :],
                                   tbufs[slot], sems[slot][1])
        cx.start()
        ct.start()
        return cx, ct

    def process(slot):
        xbuf, tbuf = xbufs[slot], tbufs[slot]
        vec_per_row = COLS // LANES
        groups_per_row = vec_per_row // UNROLL

        def vec_body(i, c2):
            r = i // groups_per_row
            cbase = (i % groups_per_row) * (UNROLL * LANES)
            xs = [None] * UNROLL
            ts = [None] * UNROLL
            for u in range(UNROLL):
                o = cbase + u * LANES
                xs[u] = xbuf[r, pl.ds(o, LANES)]
                ts[u] = tbuf[r, pl.ds(o, LANES)]
            es = [jnp.exp(xs[u]) for u in range(UNROLL)]
            ds = [1.0 + es[u] for u in range(UNROLL)]
            qs = [1.0 / ds[u] for u in range(UNROLL)]
            bs = [
                (qs[u] * (K_BINS - 0.5)).astype(jnp.int32) + ts[u] * K_BINS
                for u in range(UNROLL)
            ]
            for u in range(UNROLL):
                plsc.addupdate_scatter(cnt_v, [bs[u]], ones)
            return c2

        lax.fori_loop(0, CHUNK_ROWS * groups_per_row, vec_body, 0)

    pending = start(0, 0)
    for ci in range(NCHUNKS):
        slot = ci % 2
        pending[0].wait()
        pending[1].wait()
        if ci + 1 < NCHUNKS:
            pending = start(ci + 1, 1 - slot)
        process(slot)

    pltpu.sync_copy(cnt_v, cnt_hbm.at[wid])


def _phase1(x_flat, t_flat):
    mesh = plsc.VectorSubcoreMesh(core_axis_name="c", subcore_axis_name="s")
    f = functools.partial(
        pl.kernel,
        mesh=mesh,
        compiler_params=pltpu.CompilerParams(needs_layout_passes=False,
                                             use_tc_tiling_on_sc=True),
        out_type=[
            jax.ShapeDtypeStruct((NUM_WORKERS, NBINS2), jnp.float32),
        ],
        scratch_types=[
            pltpu.VMEM((CHUNK_ROWS, COLS), jnp.float32),
            pltpu.VMEM((CHUNK_ROWS, COLS), jnp.int32),
            pltpu.VMEM((CHUNK_ROWS, COLS), jnp.float32),
            pltpu.VMEM((CHUNK_ROWS, COLS), jnp.int32),
            pltpu.VMEM((NBINS2,), jnp.float32),
            pltpu.SemaphoreType.DMA,
            pltpu.SemaphoreType.DMA,
            pltpu.SemaphoreType.DMA,
            pltpu.SemaphoreType.DMA,
        ],
    )(_phase1_body)
    return f(x_flat, t_flat)


def _phase2_body(cnt_ref, out_ref):
    cnt = jnp.sum(cnt_ref[...], axis=0)

    rr = lax.broadcasted_iota(jnp.int32, (128, 128), 0)
    cc = lax.broadcasted_iota(jnp.int32, (128, 128), 1)
    g = rr * 128 + cc
    b = jnp.where(g >= K_BINS, g - K_BINS, g).astype(jnp.float32)
    mid = 1.0 - (b + 0.5) * (1.0 / (K_BINS - 0.5))
    sm = cnt * mid

    neg_row = (rr < 64).astype(jnp.float32)
    tri_incl = (rr <= cc).astype(jnp.float32)
    tri_strict = (cc < rr).astype(jnp.float32)

    cnt_neg = cnt * neg_row
    pos = jnp.sum(cnt * (1.0 - neg_row))
    s1 = jnp.sum(sm * (1.0 - neg_row))

    within = jnp.dot(cnt_neg, tri_incl, preferred_element_type=jnp.float32)
    prev_rows = jnp.dot(tri_strict, cnt_neg, preferred_element_type=jnp.float32)
    row_off = jnp.sum(prev_rows, axis=1, keepdims=True)
    csum = within + row_off

    rank = csum - cnt_neg
    contrib = sm * neg_row * pos / ((pos + rank) * (pos + rank + cnt_neg))
    loss = 1.0 - s1 / N_TOTAL + jnp.sum(contrib)
    out_ref[0, 0] = jnp.where(pos > 0, loss, 0.0)


def _phase2(cnt_part, interpret=False):
    return pl.pallas_call(
        _phase2_body,
        out_shape=jax.ShapeDtypeStruct((1, 1), jnp.float32),
        out_specs=pl.BlockSpec(memory_space=pltpu.SMEM),
        interpret=interpret,
    )(cnt_part)


def kernel(input, target):
    x = input.reshape(ROWS, COLS)
    t = target.reshape(ROWS, COLS).astype(jnp.int32)
    (cnt_part,) = _phase1(x, t)
    out = _phase2(cnt_part.reshape(NUM_WORKERS, 128, 128))
    return out[0, 0]


# speedup vs baseline: 121.3773x; 1.1585x over previous
import functools

import jax
import jax.numpy as jnp
from jax import lax
from jax.experimental import pallas as pl
from jax.experimental.pallas import tpu as pltpu
from jax.experimental.pallas import tpu_sc as plsc

N_TOTAL = 16 * 1 * 512 * 512
K_BINS = 8192
NBINS2 = 2 * K_BINS
GRID_R = NBINS2 // 128
NEG_R = K_BINS // 128
NUM_WORKERS = 32
ROWS = 8192
COLS = 512
ROWS_PER_TILE = ROWS // NUM_WORKERS
CHUNK_ROWS = 16
CHUNK = CHUNK_ROWS * COLS
NCHUNKS = ROWS_PER_TILE // CHUNK_ROWS
LANES = 16
UNROLL = 16


def _phase1_body(x_hbm, t_hbm, cnt_hbm,
                 xbuf0, tbuf0, xbuf1, tbuf1, cnt_v,
                 sem_x0, sem_t0, sem_x1, sem_t1):
    wid = lax.axis_index("s") * 2 + lax.axis_index("c")
    base = wid * ROWS_PER_TILE

    zeros = jnp.zeros((LANES,), jnp.float32)

    def zero_body(i, c):
        cnt_v[pl.ds(i * LANES, LANES)] = zeros
        return c

    lax.fori_loop(0, NBINS2 // LANES, zero_body, 0)

    ones = jnp.ones((LANES,), jnp.float32)
    xbufs = (xbuf0, xbuf1)
    tbufs = (tbuf0, tbuf1)
    sems = ((sem_x0, sem_t0), (sem_x1, sem_t1))

    def start(ci, slot):
        row0 = base + ci * CHUNK_ROWS
        cx = pltpu.make_async_copy(x_hbm.at[pl.ds(row0, CHUNK_ROWS), :],
                                   xbufs[slot], sems[slot][0])
        ct = pltpu.make_async_copy(t_hbm.at[pl.ds(row0, CHUNK_ROWS), :],
                                   tbufs[slot], sems[slot][1])
        cx.start()
        ct.start()
        return cx, ct

    def process(slot):
        xbuf, tbuf = xbufs[slot], tbufs[slot]
        vec_per_row = COLS // LANES
        groups_per_row = vec_per_row // UNROLL

        def vec_body(i, c2):
            r = i // groups_per_row
            cbase = (i % groups_per_row) * (UNROLL * LANES)
            xs = [None] * UNROLL
            ts = [None] * UNROLL
            for u in range(UNROLL):
                o = cbase + u * LANES
                xs[u] = xbuf[r, pl.ds(o, LANES)]
                ts[u] = tbuf[r, pl.ds(o, LANES)]
            es = [jnp.exp(xs[u]) for u in range(UNROLL)]
            ds = [1.0 + es[u] for u in range(UNROLL)]
            qs = [1.0 / ds[u] for u in range(UNROLL)]
            bs = [
                (qs[u] * (K_BINS - 0.5)).astype(jnp.int32) + ts[u] * K_BINS
                for u in range(UNROLL)
            ]
            for u in range(UNROLL):
                plsc.addupdate_scatter(cnt_v, [bs[u]], ones)
            return c2

        lax.fori_loop(0, CHUNK_ROWS * groups_per_row, vec_body, 0)

    pending = start(0, 0)
    for ci in range(NCHUNKS):
        slot = ci % 2
        pending[0].wait()
        pending[1].wait()
        if ci + 1 < NCHUNKS:
            pending = start(ci + 1, 1 - slot)
        process(slot)

    pltpu.sync_copy(cnt_v, cnt_hbm.at[wid])


def _phase1(x_flat, t_flat):
    mesh = plsc.VectorSubcoreMesh(core_axis_name="c", subcore_axis_name="s")
    f = functools.partial(
        pl.kernel,
        mesh=mesh,
        compiler_params=pltpu.CompilerParams(needs_layout_passes=False,
                                             use_tc_tiling_on_sc=True),
        out_type=[
            jax.ShapeDtypeStruct((NUM_WORKERS, NBINS2), jnp.float32),
        ],
        scratch_types=[
            pltpu.VMEM((CHUNK_ROWS, COLS), jnp.float32),
            pltpu.VMEM((CHUNK_ROWS, COLS), jnp.int32),
            pltpu.VMEM((CHUNK_ROWS, COLS), jnp.float32),
            pltpu.VMEM((CHUNK_ROWS, COLS), jnp.int32),
            pltpu.VMEM((NBINS2,), jnp.float32),
            pltpu.SemaphoreType.DMA,
            pltpu.SemaphoreType.DMA,
            pltpu.SemaphoreType.DMA,
            pltpu.SemaphoreType.DMA,
        ],
    )(_phase1_body)
    return f(x_flat, t_flat)


def _phase2_body(cnt_ref, out_ref):
    cnt = jnp.sum(cnt_ref[...], axis=0)

    rr = lax.broadcasted_iota(jnp.int32, (GRID_R, 128), 0)
    cc = lax.broadcasted_iota(jnp.int32, (GRID_R, 128), 1)
    g = rr * 128 + cc
    b = jnp.where(g >= K_BINS, g - K_BINS, g).astype(jnp.float32)
    mid = 1.0 - (b + 0.5) * (1.0 / (K_BINS - 0.5))
    sm = cnt * mid

    neg_row = (rr < NEG_R).astype(jnp.float32)
    ci = lax.broadcasted_iota(jnp.int32, (128, 128), 0)
    cj = lax.broadcasted_iota(jnp.int32, (128, 128), 1)
    tri_incl = (ci <= cj).astype(jnp.float32)
    ri = lax.broadcasted_iota(jnp.int32, (GRID_R, GRID_R), 0)
    rj = lax.broadcasted_iota(jnp.int32, (GRID_R, GRID_R), 1)
    tri_strict = (rj < ri).astype(jnp.float32)

    cnt_neg = cnt * neg_row
    pos = jnp.sum(cnt * (1.0 - neg_row))
    s1 = jnp.sum(sm * (1.0 - neg_row))

    within = jnp.dot(cnt_neg, tri_incl, preferred_element_type=jnp.float32)
    prev_rows = jnp.dot(tri_strict, cnt_neg, preferred_element_type=jnp.float32)
    row_off = jnp.sum(prev_rows, axis=1, keepdims=True)
    csum = within + row_off

    rank = csum - cnt_neg
    contrib = sm * neg_row * pos / ((pos + rank) * (pos + rank + cnt_neg))
    loss = 1.0 - s1 / N_TOTAL + jnp.sum(contrib)
    out_ref[0, 0] = jnp.where(pos > 0, loss, 0.0)


def _phase2(cnt_part, interpret=False):
    return pl.pallas_call(
        _phase2_body,
        out_shape=jax.ShapeDtypeStruct((1, 1), jnp.float32),
        out_specs=pl.BlockSpec(memory_space=pltpu.SMEM),
        interpret=interpret,
    )(cnt_part)


def kernel(input, target):
    x = input.reshape(ROWS, COLS)
    t = target.reshape(ROWS, COLS).astype(jnp.int32)
    (cnt_part,) = _phase1(x, t)
    out = _phase2(cnt_part.reshape(NUM_WORKERS, 128, 128))
    return out[0, 0]


# speedup vs baseline: 129.2482x; 1.0648x over previous
import functools

import jax
import jax.numpy as jnp
from jax import lax
from jax.experimental import pallas as pl
from jax.experimental.pallas import tpu as pltpu
from jax.experimental.pallas import tpu_sc as plsc

N_TOTAL = 16 * 1 * 512 * 512
K_BINS = 8192
NBINS2 = 2 * K_BINS
GRID_R = NBINS2 // 128
NEG_R = K_BINS // 128
NUM_WORKERS = 32
ROWS = 8192
COLS = 512
ROWS_PER_TILE = ROWS // NUM_WORKERS
CHUNK_ROWS = 16
CHUNK = CHUNK_ROWS * COLS
NCHUNKS = ROWS_PER_TILE // CHUNK_ROWS
LANES = 16
UNROLL = 32


def _phase1_body(x_hbm, t_hbm, cnt_hbm,
                 xbuf0, tbuf0, xbuf1, tbuf1, cnt_v,
                 sem_x0, sem_t0, sem_x1, sem_t1):
    wid = lax.axis_index("s") * 2 + lax.axis_index("c")
    base = wid * ROWS_PER_TILE

    zeros = jnp.zeros((LANES,), jnp.float32)

    def zero_body(i, c):
        cnt_v[pl.ds(i * LANES, LANES)] = zeros
        return c

    lax.fori_loop(0, NBINS2 // LANES, zero_body, 0)

    ones = jnp.ones((LANES,), jnp.float32)
    xbufs = (xbuf0, xbuf1)
    tbufs = (tbuf0, tbuf1)
    sems = ((sem_x0, sem_t0), (sem_x1, sem_t1))

    def start(ci, slot):
        row0 = base + ci * CHUNK_ROWS
        cx = pltpu.make_async_copy(x_hbm.at[pl.ds(row0, CHUNK_ROWS), :],
                                   xbufs[slot], sems[slot][0])
        ct = pltpu.make_async_copy(t_hbm.at[pl.ds(row0, CHUNK_ROWS), :],
                                   tbufs[slot], sems[slot][1])
        cx.start()
        ct.start()
        return cx, ct

    def process(slot):
        xbuf, tbuf = xbufs[slot], tbufs[slot]
        vec_per_row = COLS // LANES
        groups_per_row = vec_per_row // UNROLL

        def vec_body(i, c2):
            r = i // groups_per_row
            cbase = (i % groups_per_row) * (UNROLL * LANES)
            xs = [None] * UNROLL
            ts = [None] * UNROLL
            for u in range(UNROLL):
                o = cbase + u * LANES
                xs[u] = xbuf[r, pl.ds(o, LANES)]
                ts[u] = tbuf[r, pl.ds(o, LANES)]
            es = [jnp.exp(xs[u]) for u in range(UNROLL)]
            ds = [1.0 + es[u] for u in range(UNROLL)]
            qs = [1.0 / ds[u] for u in range(UNROLL)]
            bs = [
                (qs[u] * (K_BINS - 0.5)).astype(jnp.int32) + ts[u] * K_BINS
                for u in range(UNROLL)
            ]
            for u in range(UNROLL):
                plsc.addupdate_scatter(cnt_v, [bs[u]], ones)
            return c2

        lax.fori_loop(0, CHUNK_ROWS * groups_per_row, vec_body, 0)

    pending = start(0, 0)
    for ci in range(NCHUNKS):
        slot = ci % 2
        pending[0].wait()
        pending[1].wait()
        if ci + 1 < NCHUNKS:
            pending = start(ci + 1, 1 - slot)
        process(slot)

    pltpu.sync_copy(cnt_v, cnt_hbm.at[wid])


def _phase1(x_flat, t_flat):
    mesh = plsc.VectorSubcoreMesh(core_axis_name="c", subcore_axis_name="s")
    f = functools.partial(
        pl.kernel,
        mesh=mesh,
        compiler_params=pltpu.CompilerParams(needs_layout_passes=False,
                                             use_tc_tiling_on_sc=True),
        out_type=[
            jax.ShapeDtypeStruct((NUM_WORKERS, NBINS2), jnp.float32),
        ],
        scratch_types=[
            pltpu.VMEM((CHUNK_ROWS, COLS), jnp.float32),
            pltpu.VMEM((CHUNK_ROWS, COLS), jnp.int32),
            pltpu.VMEM((CHUNK_ROWS, COLS), jnp.float32),
            pltpu.VMEM((CHUNK_ROWS, COLS), jnp.int32),
            pltpu.VMEM((NBINS2,), jnp.float32),
            pltpu.SemaphoreType.DMA,
            pltpu.SemaphoreType.DMA,
            pltpu.SemaphoreType.DMA,
            pltpu.SemaphoreType.DMA,
        ],
    )(_phase1_body)
    return f(x_flat, t_flat)


def _phase2_body(cnt_ref, out_ref):
    cnt = jnp.sum(cnt_ref[...], axis=0)

    rr = lax.broadcasted_iota(jnp.int32, (GRID_R, 128), 0)
    cc = lax.broadcasted_iota(jnp.int32, (GRID_R, 128), 1)
    g = rr * 128 + cc
    b = jnp.where(g >= K_BINS, g - K_BINS, g).astype(jnp.float32)
    mid = 1.0 - (b + 0.5) * (1.0 / (K_BINS - 0.5))
    sm = cnt * mid

    neg_row = (rr < NEG_R).astype(jnp.float32)
    ci = lax.broadcasted_iota(jnp.int32, (128, 128), 0)
    cj = lax.broadcasted_iota(jnp.int32, (128, 128), 1)
    tri_incl = (ci <= cj).astype(jnp.float32)
    ri = lax.broadcasted_iota(jnp.int32, (GRID_R, GRID_R), 0)
    rj = lax.broadcasted_iota(jnp.int32, (GRID_R, GRID_R), 1)
    tri_strict = (rj < ri).astype(jnp.float32)

    cnt_neg = cnt * neg_row
    pos = jnp.sum(cnt * (1.0 - neg_row))
    s1 = jnp.sum(sm * (1.0 - neg_row))

    within = jnp.dot(cnt_neg, tri_incl, preferred_element_type=jnp.float32)
    prev_rows = jnp.dot(tri_strict, cnt_neg, preferred_element_type=jnp.float32)
    row_off = jnp.sum(prev_rows, axis=1, keepdims=True)
    csum = within + row_off

    rank = csum - cnt_neg
    contrib = sm * neg_row * pos / ((pos + rank) * (pos + rank + cnt_neg))
    loss = 1.0 - s1 / N_TOTAL + jnp.sum(contrib)
    out_ref[0, 0] = jnp.where(pos > 0, loss, 0.0)


def _phase2(cnt_part, interpret=False):
    return pl.pallas_call(
        _phase2_body,
        out_shape=jax.ShapeDtypeStruct((1, 1), jnp.float32),
        out_specs=pl.BlockSpec(memory_space=pltpu.SMEM),
        interpret=interpret,
    )(cnt_part)


def kernel(input, target):
    x = input.reshape(ROWS, COLS)
    t = target.reshape(ROWS, COLS).astype(jnp.int32)
    (cnt_part,) = _phase1(x, t)
    out = _phase2(cnt_part.reshape(NUM_WORKERS, 128, 128))
    return out[0, 0]
